# baseline (speedup 1.0000x reference)
import functools

import jax
import jax.numpy as jnp
from jax import lax
from jax.experimental import pallas as pl

_E = 64


def _kmoe_block_kernel(x_ref, l_ref, ab_ref, scale_ref, bias_ref,
                       o_ref, *, di1, di2, do1, do2, transposed, pre_gelu, out_3d):
    tb = x_ref.shape[0]
    lg = l_ref[...]
    m1 = jnp.max(lg, axis=-1)
    i1 = jnp.argmax(lg, axis=-1)
    eids = lax.broadcasted_iota(jnp.int32, (tb, _E), 1)
    masked = jnp.where(eids == i1[:, None], -jnp.inf, lg)
    m2 = jnp.max(masked, axis=-1)
    i2 = jnp.argmax(masked, axis=-1)
    e = jnp.exp(m2 - m1)
    den = 1.0 + e
    p1 = 1.0 / den
    p2 = e / den

    xv = x_ref[...]
    if pre_gelu:
        xv = 0.5 * xv * (1.0 - lax.erf(-xv * 0.7071067811865476))
    x3 = xv.astype(jnp.bfloat16).reshape(tb, di1, di2)

    oh2 = (jnp.concatenate([i1, i2])[:, None] ==
           lax.broadcasted_iota(jnp.int32, (2 * tb, _E), 1)
           ).astype(jnp.bfloat16)
    g = jnp.dot(oh2, ab_ref[...],
                preferred_element_type=jnp.float32
                ).astype(jnp.bfloat16)
    acc = jnp.zeros((tb, do1, do2), dtype=jnp.float32)
    if transposed:
        g3 = g.reshape(2 * tb, di1, do1 + do2)
        for k, prob in ((0, p1), (1, p2)):
            ag = g3[k * tb:(k + 1) * tb, :, :do1]
            bg = g3[k * tb:(k + 1) * tb, :, do1:]
            t1 = lax.dot_general(ag, x3, (((1,), (1,)), ((0,), (0,))),
                                 preferred_element_type=jnp.float32)
            y = lax.dot_general(t1.astype(jnp.bfloat16), bg,
                                (((2,), (1,)), ((0,), (0,))),
                                preferred_element_type=jnp.float32)
            acc = acc + y * prob[:, None, None]
    else:
        g3 = g.reshape(2 * tb, do1 + do2, di1)
        for k, prob in ((0, p1), (1, p2)):
            ag = g3[k * tb:(k + 1) * tb, :do1, :]
            bg = g3[k * tb:(k + 1) * tb, do1:, :]
            t1 = lax.dot_general(ag, x3, (((2,), (1,)), ((0,), (0,))),
                                 preferred_element_type=jnp.float32)
            y = lax.dot_general(t1.astype(jnp.bfloat16), bg,
                                (((2,), (2,)), ((0,), (0,))),
                                preferred_element_type=jnp.float32)
            acc = acc + y * prob[:, None, None]

    if out_3d:
        o_ref[...] = acc * scale_ref[0, 0, 0] + bias_ref[...]
    else:
        o_ref[...] = (acc.reshape(tb, do1 * do2) * scale_ref[0, 0]
                      + bias_ref[0, :])


def _kmoe_layer(x_flat, logits, a, b, scale, bias, di1, di2, do1, do2,
                transposed=False, pre_gelu=False, out_3d=False, tb=256, interpret=False):
    n = x_flat.shape[0]
    d_in = di1 * di2
    d_out = do1 * do2
    if transposed:
        ab2 = jnp.concatenate([jnp.transpose(a, (0, 2, 1)),
                               jnp.transpose(b, (0, 2, 1))], axis=2).reshape(
            _E, di1 * (do1 + do2)).astype(jnp.bfloat16)
    else:
        ab2 = jnp.concatenate([a.reshape(_E, do1 * di1),
                               b.reshape(_E, do2 * di2)],
                              axis=1).astype(jnp.bfloat16)
    if out_3d:
        scale2 = scale.reshape(1, 1, 1)
        bias2 = bias.reshape(1, do1, do2)
        sb_specs = [pl.BlockSpec((1, 1, 1), lambda i: (0, 0, 0)),
                    pl.BlockSpec((1, do1, do2), lambda i: (0, 0, 0))]
        out_spec = pl.BlockSpec((tb, do1, do2), lambda i: (i, 0, 0))
        out_shape = jax.ShapeDtypeStruct((n, do1, do2), jnp.float32)
    else:
        scale2 = scale.reshape(1, 1)
        bias2 = bias.reshape(1, d_out)
        sb_specs = [pl.BlockSpec((1, 1), lambda i: (0, 0)),
                    pl.BlockSpec((1, d_out), lambda i: (0, 0))]
        out_spec = pl.BlockSpec((tb, d_out), lambda i: (i, 0))
        out_shape = jax.ShapeDtypeStruct((n, d_out), jnp.float32)
    grid = (n // tb,)
    return pl.pallas_call(
        functools.partial(_kmoe_block_kernel, di1=di1, di2=di2,
                          do1=do1, do2=do2, transposed=transposed, pre_gelu=pre_gelu, out_3d=out_3d),
        grid=grid,
        in_specs=[
            pl.BlockSpec((tb, d_in), lambda i: (i, 0)),
            pl.BlockSpec((tb, _E), lambda i: (i, 0)),
            pl.BlockSpec((_E, (do1 + do2) * di1), lambda i: (0, 0)),
        ] + sb_specs,
        out_specs=out_spec,
        out_shape=out_shape,
        interpret=interpret,
    )(x_flat, logits, ab2, scale2, bias2)


def kernel(x, router_up, A_up, B_up, scale_up, bias_up,
           router_down, A_down, B_down, scale_down, bias_down,
           interpret=False):
    orig_shape = x.shape
    n = x.size // 1024
    x_flat = x.reshape(-1, 1024)
    logits_up = x_flat @ router_up.T
    u = _kmoe_layer(x_flat, logits_up, A_up, B_up, scale_up, bias_up,
                    32, 32, 64, 64, transposed=True, out_3d=True,
                    interpret=interpret).reshape(n, 4096)
    logits_down = jax.nn.gelu(u, approximate=False) @ router_down.T
    y = _kmoe_layer(u, logits_down, A_down, B_down, scale_down, bias_down,
                    64, 64, 32, 32, pre_gelu=True, tb=512, interpret=interpret)
    return y.reshape(orig_shape[:-1] + (1024,))


# speedup vs baseline: 1.2899x; 1.2899x over previous
import functools

import jax
import jax.numpy as jnp
from jax import lax
from jax.experimental import pallas as pl

_E = 64


def _kmoe_block_kernel(x_ref, l_ref, ab_ref, scale_ref, bias_ref,
                       o_ref, *, di1, di2, do1, do2, transposed, pre_gelu, out_3d):
    tb = x_ref.shape[0]
    lg = l_ref[...]
    m1 = jnp.max(lg, axis=-1)
    i1 = jnp.argmax(lg, axis=-1)
    eids = lax.broadcasted_iota(jnp.int32, (tb, _E), 1)
    masked = jnp.where(eids == i1[:, None], -jnp.inf, lg)
    m2 = jnp.max(masked, axis=-1)
    i2 = jnp.argmax(masked, axis=-1)
    e = jnp.exp(m2 - m1)
    den = 1.0 + e
    p1 = 1.0 / den
    p2 = e / den

    xv = x_ref[...]
    if pre_gelu:
        xv = 0.5 * xv * (1.0 - lax.erf(-xv * 0.7071067811865476))
    x3 = xv.astype(jnp.bfloat16).reshape(tb, di1, di2)

    oh2 = (jnp.concatenate([i1, i2])[:, None] ==
           lax.broadcasted_iota(jnp.int32, (2 * tb, _E), 1)
           ).astype(jnp.bfloat16)
    g = jnp.dot(oh2, ab_ref[...],
                preferred_element_type=jnp.float32
                ).astype(jnp.bfloat16)
    acc = jnp.zeros((tb, do1, do2), dtype=jnp.float32)
    if transposed:
        g3 = g.reshape(2 * tb, di1, do1 + do2)
        for k, prob in ((0, p1), (1, p2)):
            ag = g3[k * tb:(k + 1) * tb, :, :do1]
            bg = g3[k * tb:(k + 1) * tb, :, do1:]
            t1 = lax.dot_general(ag, x3, (((1,), (1,)), ((0,), (0,))),
                                 preferred_element_type=jnp.float32)
            y = lax.dot_general(t1, bg.astype(jnp.float32),
                                (((2,), (1,)), ((0,), (0,))),
                                preferred_element_type=jnp.float32)
            acc = acc + y * prob[:, None, None]
    else:
        g3 = g.reshape(2 * tb, do1 + do2, di1)
        for k, prob in ((0, p1), (1, p2)):
            ag = g3[k * tb:(k + 1) * tb, :do1, :]
            bg = g3[k * tb:(k + 1) * tb, do1:, :]
            t1 = lax.dot_general(ag, x3, (((2,), (1,)), ((0,), (0,))),
                                 preferred_element_type=jnp.float32)
            y = lax.dot_general(t1, bg.astype(jnp.float32),
                                (((2,), (2,)), ((0,), (0,))),
                                preferred_element_type=jnp.float32)
            acc = acc + y * prob[:, None, None]

    if out_3d:
        o_ref[...] = acc * scale_ref[0, 0, 0] + bias_ref[...]
    else:
        o_ref[...] = (acc.reshape(tb, do1 * do2) * scale_ref[0, 0]
                      + bias_ref[0, :])


def _kmoe_layer(x_flat, logits, a, b, scale, bias, di1, di2, do1, do2,
                transposed=False, pre_gelu=False, out_3d=False, tb=256, interpret=False):
    n = x_flat.shape[0]
    d_in = di1 * di2
    d_out = do1 * do2
    if transposed:
        ab2 = jnp.concatenate([jnp.transpose(a, (0, 2, 1)),
                               jnp.transpose(b, (0, 2, 1))], axis=2).reshape(
            _E, di1 * (do1 + do2)).astype(jnp.bfloat16)
    else:
        ab2 = jnp.concatenate([a.reshape(_E, do1 * di1),
                               b.reshape(_E, do2 * di2)],
                              axis=1).astype(jnp.bfloat16)
    if out_3d:
        scale2 = scale.reshape(1, 1, 1)
        bias2 = bias.reshape(1, do1, do2)
        sb_specs = [pl.BlockSpec((1, 1, 1), lambda i: (0, 0, 0)),
                    pl.BlockSpec((1, do1, do2), lambda i: (0, 0, 0))]
        out_spec = pl.BlockSpec((tb, do1, do2), lambda i: (i, 0, 0))
        out_shape = jax.ShapeDtypeStruct((n, do1, do2), jnp.float32)
    else:
        scale2 = scale.reshape(1, 1)
        bias2 = bias.reshape(1, d_out)
        sb_specs = [pl.BlockSpec((1, 1), lambda i: (0, 0)),
                    pl.BlockSpec((1, d_out), lambda i: (0, 0))]
        out_spec = pl.BlockSpec((tb, d_out), lambda i: (i, 0))
        out_shape = jax.ShapeDtypeStruct((n, d_out), jnp.float32)
    grid = (n // tb,)
    return pl.pallas_call(
        functools.partial(_kmoe_block_kernel, di1=di1, di2=di2,
                          do1=do1, do2=do2, transposed=transposed, pre_gelu=pre_gelu, out_3d=out_3d),
        grid=grid,
        in_specs=[
            pl.BlockSpec((tb, d_in), lambda i: (i, 0)),
            pl.BlockSpec((tb, _E), lambda i: (i, 0)),
            pl.BlockSpec((_E, (do1 + do2) * di1), lambda i: (0, 0)),
        ] + sb_specs,
        out_specs=out_spec,
        out_shape=out_shape,
        interpret=interpret,
    )(x_flat, logits, ab2, scale2, bias2)


def kernel(x, router_up, A_up, B_up, scale_up, bias_up,
           router_down, A_down, B_down, scale_down, bias_down,
           interpret=False):
    orig_shape = x.shape
    n = x.size // 1024
    x_flat = x.reshape(-1, 1024)
    logits_up = x_flat @ router_up.T
    u = _kmoe_layer(x_flat, logits_up, A_up, B_up, scale_up, bias_up,
                    32, 32, 64, 64, transposed=True,
                    interpret=interpret).reshape(n, 4096)
    logits_down = jax.nn.gelu(u, approximate=False) @ router_down.T
    y = _kmoe_layer(u, logits_down, A_down, B_down, scale_down, bias_down,
                    64, 64, 32, 32, pre_gelu=True, tb=512, interpret=interpret)
    return y.reshape(orig_shape[:-1] + (1024,))
